# Initial kernel scaffold; baseline (speedup 1.0000x reference)
import jax
import jax.numpy as jnp
from jax.experimental import pallas as pl


def kernel(state, action, edge_index, agent_i, Wg, bg, W1, b1, g1, be1, W2, b2, g2, be2, Wa, ba, Wq, bq):
    raise NotImplementedError("write your pallas kernel here")


# speedup vs baseline: 51.8474x; 51.8474x over previous
import functools

import jax
import jax.numpy as jnp
from jax import lax
from jax.experimental import pallas as pl
from jax.experimental.pallas import tpu as pltpu
from jax.experimental.pallas import tpu_sc as plsc

N = 10000
E = 320000
D = 128
H = 256
B = 1024
EPS = 1e-5

L = 16
NCORES = 2
NSUB = 16
NROWS = 640
NP = NROWS * L
NSEL = NP
EPT1 = E // NSUB
EPT2 = E // (NCORES * NSUB)
ACC_ROWS = B + L
BPT = B // NSUB


def _rsqrt16(x):
    xh = 0.5 * x
    i = plsc.bitcast(x, jnp.int32)
    i = 0x5F3759DF - (i >> 1)
    y = plsc.bitcast(i, jnp.float32)
    y = y * (1.5 - xh * y * y)
    y = y * (1.5 - xh * y * y)
    y = y * (1.5 - xh * y * y)
    return y


def _sc_body(src_hbm, dst_hbm, ag_hbm, xw_hbm, out_hbm,
             ag_v, n2s_v, dinv_v, hist_v, dst_v, src_v,
             ssel_v, lsel_v, nsel_v, idxm_v, gbuf_v, gidx_v, sidx_v,
             obuf_v, oa_v, os_v, degS, accS, sem):
    cid = lax.axis_index("c")
    sid = lax.axis_index("s")
    wid = sid * NCORES + cid
    iota = lax.iota(jnp.int32, L)
    zf = jnp.zeros((L,), jnp.float32)
    zi = jnp.zeros((L,), jnp.int32)
    ones = jnp.ones((L,), jnp.float32)

    def z_hist(i, c):
        hist_v[i, :] = zf
        return c
    lax.fori_loop(0, NROWS, z_hist, 0)

    def z_misc(q, c):
        n2s_v[pl.ds(q * L, L)] = jnp.full((L,), -1, jnp.int32)
        ssel_v[pl.ds(q * L, L)] = zi
        lsel_v[pl.ds(q * L, L)] = jnp.full((L,), B, jnp.int32)
        nsel_v[pl.ds(q * L, L)] = zf
        return c
    lax.fori_loop(0, NROWS, z_misc, 0)

    def z_gbuf(r, c):
        for j in range(H // L):
            gbuf_v[r, pl.ds(j * L, L)] = zf
        return c
    lax.fori_loop(0, L, z_gbuf, 0)

    for j in range(5):
        for i in range(8):
            idxm_v[j, pl.ds(i * L, L)] = j * 128 + i * L + iota

    pltpu.sync_copy(ag_hbm, ag_v)

    def b_n2s(q, c):
        a16 = ag_v[pl.ds(q * L, L)]
        plsc.store_scatter(n2s_v, [a16], q * L + iota)
        return c
    lax.fori_loop(0, B // L, b_n2s, 0)

    @pl.when(sid == 0)
    def _():
        pltpu.sync_copy(hist_v, degS)

    for j in range(4):
        pltpu.sync_copy(gbuf_v, accS.at[pl.ds(sid * 64 + j * L, L)])

    @pl.when(sid == 0)
    def _():
        pltpu.sync_copy(gbuf_v, accS.at[pl.ds(B, L)])

    plsc.subcore_barrier()

    pltpu.sync_copy(dst_hbm.at[pl.ds(sid * EPT1, EPT1)], dst_v)

    def hist_step(k, c):
        d16 = dst_v[pl.ds(k * L, L)]
        plsc.addupdate_scatter(hist_v, [d16 >> 4, d16 & 15], ones)
        return c
    lax.fori_loop(0, EPT1 // L, hist_step, 0)

    for j in range(5):
        pltpu.sync_copy(hist_v.at[pl.ds(j * 128, 128)],
                        degS.at[idxm_v.at[j]], add=True)
    plsc.subcore_barrier()

    pltpu.sync_copy(degS, hist_v)

    def dinv_step(i, c):
        deg = hist_v[i, :] + 1.0
        dinv_v[pl.ds(i * L, L)] = _rsqrt16(deg)
        return c
    lax.fori_loop(0, NROWS, dinv_step, 0)

    pltpu.sync_copy(src_hbm.at[pl.ds(wid * EPT2, EPT2)], src_v)
    pltpu.sync_copy(dst_hbm.at[pl.ds(wid * EPT2, EPT2)], dst_v.at[pl.ds(0, EPT2)])

    def filt(k, n):
        s16 = src_v[pl.ds(k * L, L)]
        d16 = dst_v[pl.ds(k * L, L)]
        slot = plsc.load_gather(n2s_v, [d16])
        m = slot >= 0
        nr = plsc.load_gather(dinv_v, [s16]) * plsc.load_gather(dinv_v, [d16])
        plsc.store_compressed(ssel_v.at[pl.ds(n, L)], s16, mask=m)
        plsc.store_compressed(lsel_v.at[pl.ds(n, L)], slot, mask=m)
        plsc.store_compressed(nsel_v.at[pl.ds(n, L)], nr, mask=m)
        cnt = plsc.all_reduce_population_count(m)
        return n + cnt[0]
    nvalid = lax.fori_loop(0, EPT2 // L, filt, jnp.int32(0))

    nch = (nvalid + L - 1) // L

    def msg(j, c):
        gidx_v[...] = ssel_v[pl.ds(j * L, L)]
        sidx_v[...] = lsel_v[pl.ds(j * L, L)]
        nv16 = nsel_v[pl.ds(j * L, L)]
        pltpu.async_copy(xw_hbm.at[gidx_v], gbuf_v, sem).wait()
        for r in range(L):
            nv = nv16[r]
            for cc in range(H // L):
                gbuf_v[r, pl.ds(cc * L, L)] = gbuf_v[r, pl.ds(cc * L, L)] * nv
        pltpu.sync_copy(gbuf_v, accS.at[sidx_v], add=True)
        return c
    lax.fori_loop(0, nch, msg, 0)
    plsc.subcore_barrier()

    boff = sid * BPT
    for q in range(BPT // L):
        a16 = ag_v[pl.ds(boff + q * L, L)]
        oa_v[pl.ds(q * L, L)] = a16
        os_v[pl.ds(q * L, L)] = plsc.load_gather(n2s_v, [a16])
    pltpu.async_copy(accS.at[os_v], obuf_v, sem).wait()

    @pl.when(cid == 0)
    def _():
        def selfq(q, c):
            a16 = oa_v[pl.ds(q * L, L)]
            gidx_v[...] = a16
            dv16 = plsc.load_gather(dinv_v, [a16])
            dv2_16 = dv16 * dv16
            pltpu.async_copy(xw_hbm.at[gidx_v], gbuf_v, sem).wait()
            for r in range(L):
                dv2 = dv2_16[r]
                for cc in range(H // L):
                    obuf_v[q * L + r, pl.ds(cc * L, L)] = (
                        obuf_v[q * L + r, pl.ds(cc * L, L)]
                        + gbuf_v[r, pl.ds(cc * L, L)] * dv2)
            return c
        lax.fori_loop(0, BPT // L, selfq, 0)

    pltpu.sync_copy(obuf_v, out_hbm.at[cid, pl.ds(boff, BPT)])


_sc_mesh = plsc.VectorSubcoreMesh(
    core_axis_name="c", subcore_axis_name="s",
    num_cores=NCORES, num_subcores=NSUB)

_sc_gather = functools.partial(
    pl.kernel,
    out_type=jax.ShapeDtypeStruct((NCORES, B, H), jnp.float32),
    mesh=_sc_mesh,
    compiler_params=pltpu.CompilerParams(
        use_tc_tiling_on_sc=False, needs_layout_passes=False),
    scratch_types=[
        pltpu.VMEM((B,), jnp.int32),
        pltpu.VMEM((NP,), jnp.int32),
        pltpu.VMEM((NP,), jnp.float32),
        pltpu.VMEM((NROWS, L), jnp.float32),
        pltpu.VMEM((EPT1,), jnp.int32),
        pltpu.VMEM((EPT2,), jnp.int32),
        pltpu.VMEM((NSEL,), jnp.int32),
        pltpu.VMEM((NSEL,), jnp.int32),
        pltpu.VMEM((NSEL,), jnp.float32),
        pltpu.VMEM((5, 128), jnp.int32),
        pltpu.VMEM((L, H), jnp.float32),
        pltpu.VMEM((L,), jnp.int32),
        pltpu.VMEM((L,), jnp.int32),
        pltpu.VMEM((BPT, H), jnp.float32),
        pltpu.VMEM((BPT,), jnp.int32),
        pltpu.VMEM((BPT,), jnp.int32),
        pltpu.VMEM_SHARED((NROWS, L), jnp.float32),
        pltpu.VMEM_SHARED((ACC_ROWS, H), jnp.float32),
        pltpu.SemaphoreType.DMA,
    ],
)(_sc_body)


def _mm_body(x_ref, w_ref, o_ref):
    o_ref[...] = jnp.dot(x_ref[...], w_ref[...],
                         preferred_element_type=jnp.float32)


def _head_body(hp_ref, act_ref, bg_ref, w1_ref, b1_ref, g1_ref, be1_ref,
               w2_ref, b2_ref, g2_ref, be2_ref, wa_ref, ba_ref, wq_ref,
               bq_ref, o_ref):
    h = jnp.maximum(hp_ref[0] + hp_ref[1] + bg_ref[...], 0.0)
    sv = jnp.dot(h, w1_ref[...], preferred_element_type=jnp.float32) + b1_ref[...]
    mu = jnp.mean(sv, axis=-1, keepdims=True)
    sv = sv - mu
    var = jnp.mean(sv * sv, axis=-1, keepdims=True)
    sv = sv * lax.rsqrt(var + EPS) * g1_ref[...] + be1_ref[...]
    sv = jnp.maximum(sv, 0.0)
    sv = jnp.dot(sv, w2_ref[...], preferred_element_type=jnp.float32) + b2_ref[...]
    mu = jnp.mean(sv, axis=-1, keepdims=True)
    sv = sv - mu
    var = jnp.mean(sv * sv, axis=-1, keepdims=True)
    sv = sv * lax.rsqrt(var + EPS) * g2_ref[...] + be2_ref[...]
    av = jnp.dot(act_ref[...], wa_ref[...],
                 preferred_element_type=jnp.float32) + ba_ref[...]
    sav = jnp.maximum(sv + av, 0.0)
    o_ref[...] = jnp.dot(sav, wq_ref[...],
                         preferred_element_type=jnp.float32) + bq_ref[...]


def kernel(state, action, edge_index, agent_i, Wg, bg, W1, b1, g1, be1,
           W2, b2, g2, be2, Wa, ba, Wq, bq):
    xw = pl.pallas_call(
        _mm_body,
        out_shape=jax.ShapeDtypeStruct((N, H), jnp.float32),
    )(state, Wg)

    hpre2 = _sc_gather(edge_index[0], edge_index[1], agent_i, xw)

    return pl.pallas_call(
        _head_body,
        out_shape=jax.ShapeDtypeStruct((B, 1), jnp.float32),
    )(hpre2, action,
      bg.reshape(1, H), W1, b1.reshape(1, H), g1.reshape(1, H),
      be1.reshape(1, H), W2, b2.reshape(1, D), g2.reshape(1, D),
      be2.reshape(1, D), Wa, ba.reshape(1, D), Wq, bq.reshape(1, 1))
